# Initial kernel scaffold; baseline (speedup 1.0000x reference)
import jax
import jax.numpy as jnp
from jax.experimental import pallas as pl


def kernel(xyz, points, W_kernel, W_linear, W_wp, W_wc):
    raise NotImplementedError("write your pallas kernel here")


# speedup vs baseline: 1.0000x; 1.0000x over previous
import jax
import jax.numpy as jnp
from jax.experimental import pallas as pl

NPOINT = 2048
NSAMPLE = 16
IN_CH = 67
OUT_CH = 64
LEAKY = 0.1


def _leaky(x):
    return jnp.where(x >= 0, x, LEAKY * x)


def _fps(xyz, npoint):
    B, N, _ = xyz.shape
    def body(i, state):
        dist, farthest, idxs = state
        idxs = idxs.at[:, i].set(farthest)
        centroid = jnp.take_along_axis(xyz, farthest[:, None, None], axis=1)
        d = jnp.sum((xyz - centroid) ** 2, axis=-1)
        dist = jnp.minimum(dist, d)
        farthest = jnp.argmax(dist, axis=-1).astype(jnp.int32)
        return (dist, farthest, idxs)
    dist0 = jnp.full((B, N), 1e10, dtype=xyz.dtype)
    far0 = jnp.zeros((B,), dtype=jnp.int32)
    idxs0 = jnp.zeros((B, npoint), dtype=jnp.int32)
    _, _, idxs = jax.lax.fori_loop(0, npoint, body, (dist0, far0, idxs0))
    return idxs


def kernel(xyz, points, W_kernel, W_linear, W_wp, W_wc):
    xyz_t = xyz.transpose(0, 2, 1)
    pts_t = points.transpose(0, 2, 1)
    fps_idx = _fps(xyz_t, NPOINT)
    new_xyz = jnp.take_along_axis(xyz_t, fps_idx[..., None], axis=1)
    sqr = -2.0 * jnp.matmul(new_xyz, xyz_t.transpose(0, 2, 1))
    sqr = sqr + jnp.sum(new_xyz ** 2, -1)[:, :, None]
    sqr = sqr + jnp.sum(xyz_t ** 2, -1)[:, None, :]
    _, knn_idx = jax.lax.top_k(-sqr, NSAMPLE)
    B, S, K = knn_idx.shape
    g_xyz = jnp.take_along_axis(xyz_t, knn_idx.reshape(B, S * K)[..., None], axis=1).reshape(B, S, K, 3)
    g_pts = jnp.take_along_axis(pts_t, knn_idx.reshape(B, S * K)[..., None], axis=1).reshape(B, S, K, 64)
    new_points = jnp.concatenate([g_xyz - new_xyz[:, :, None, :], g_pts], axis=-1)
    x = new_points.transpose(0, 3, 1, 2)
    x = _leaky(jnp.einsum('oi,bisk->bosk', W_kernel, x))
    channel_avg = jnp.mean(x, -1)
    point_avg = jnp.mean(x, 1)
    agg = jnp.concatenate([channel_avg, point_avg.transpose(0, 2, 1)], axis=1)
    agg = _leaky(jnp.einsum('oi,bis->bos', W_linear, agg))
    weight_point = jax.nn.sigmoid(jnp.einsum('oi,bis->bos', W_wp, agg[:, -NSAMPLE:, :]))
    weight_point = weight_point.transpose(0, 2, 1)[:, None, :, :]
    weight_channel = jax.nn.sigmoid(jnp.einsum('oi,bis->bos', W_wc, agg[:, :-NSAMPLE, :]))[..., None]
    x = x * weight_channel * weight_point
    out = jnp.mean(x, -1)
    return (new_xyz.transpose(0, 2, 1), out, fps_idx)


# speedup vs baseline: 19.6441x; 19.6436x over previous
import functools

import jax
import jax.numpy as jnp
from jax import lax
from jax.experimental import pallas as pl
from jax.experimental.pallas import tpu as pltpu
from jax.experimental.pallas import tpu_sc as plsc

B = 2
N = 8192
S = 2048
K = 16
D = 64
OUT = 64
LEAKY = 0.1

_I32_MAX = 2147483647


def _leaky(x):
    return jnp.where(x >= 0, x, LEAKY * x)


def _fps_body(xyz_ref, idx_ref, nxyz_ref):
    x = xyz_ref[:, 0]
    y = xyz_ref[:, 1]
    z = xyz_ref[:, 2]
    lin = (lax.broadcasted_iota(jnp.int32, (64, 128), 0) * 128
           + lax.broadcasted_iota(jnp.int32, (64, 128), 1))[None]
    lin2 = (lax.broadcasted_iota(jnp.int32, (16, 128), 0) * 128
            + lax.broadcasted_iota(jnp.int32, (16, 128), 1))[None]

    def red2(a, op):
        return op(op(a, axis=2, keepdims=True), axis=1, keepdims=True)

    def body(i, carry):
        far, dist = carry
        m2 = lin2 == i
        idx_ref[...] = jnp.where(m2, far, idx_ref[...])
        em = lin == far
        cx = red2(jnp.where(em, x, 0.0), jnp.sum)
        cy = red2(jnp.where(em, y, 0.0), jnp.sum)
        cz = red2(jnp.where(em, z, 0.0), jnp.sum)
        nxyz_ref[:, 0] = jnp.where(m2, cx, nxyz_ref[:, 0])
        nxyz_ref[:, 1] = jnp.where(m2, cy, nxyz_ref[:, 1])
        nxyz_ref[:, 2] = jnp.where(m2, cz, nxyz_ref[:, 2])
        dx = x - cx
        dy = y - cy
        dz = z - cz
        d = dx * dx + dy * dy
        d = d + dz * dz
        dist = jnp.minimum(dist, d)
        m = red2(dist, jnp.max)
        cand = jnp.where(dist == m, lin, _I32_MAX)
        far_new = red2(cand, jnp.min)
        return far_new, dist

    far0 = jnp.zeros((B, 1, 1), jnp.int32)
    dist0 = jnp.full((B, 64, 128), 1e10, jnp.float32)
    lax.fori_loop(0, S, body, (far0, dist0))


def _fps_call(xyz):
    xyzr = xyz.reshape(B, 3, 64, 128)
    idx, nxyz = pl.pallas_call(
        _fps_body,
        out_shape=[
            jax.ShapeDtypeStruct((B, 16, 128), jnp.int32),
            jax.ShapeDtypeStruct((B, 3, 16, 128), jnp.float32),
        ],
    )(xyzr)
    return idx.reshape(B, S), nxyz.reshape(B, 3, S)


_CPAD = 128


def _proj_body(xyz_ref, pts_ref, w3_ref, wp_ref, out_ref):
    xb = xyz_ref[0]
    pb = pts_ref[0]
    pt = (lax.dot_general(w3_ref[...], xb, (((1,), (0,)), ((), ())),
                          preferred_element_type=jnp.float32)
          + lax.dot_general(wp_ref[...], pb, (((1,), (0,)), ((), ())),
                            preferred_element_type=jnp.float32))
    out_ref[...] = lax.transpose(pt, (1, 0))


def _proj_call(xyz, points, W3pad, Wppad):
    return pl.pallas_call(
        _proj_body,
        grid=(B,),
        in_specs=[
            pl.BlockSpec((1, 3, N), lambda b: (b, 0, 0)),
            pl.BlockSpec((1, D, N), lambda b: (b, 0, 0)),
            pl.BlockSpec((_CPAD, 3), lambda b: (0, 0)),
            pl.BlockSpec((_CPAD, D), lambda b: (0, 0)),
        ],
        out_specs=pl.BlockSpec((N, _CPAD), lambda b: (b, 0)),
        out_shape=jax.ShapeDtypeStruct((B * N, _CPAD), jnp.float32),
    )(xyz, points, W3pad, Wppad)


_S_TILE = 128


def _knn_body(nq_ref, xyz_ref, out_ref):
    q = nq_ref[0]
    xb = xyz_ref[0]
    d = -2.0 * lax.dot_general(q, xb, (((0,), (0,)), ((), ())),
                               preferred_element_type=jnp.float32)
    qn = lax.dot_general(q * q, jnp.ones((3, 1), jnp.float32),
                         (((0,), (0,)), ((), ())),
                         preferred_element_type=jnp.float32)
    d = d + qn
    xn = jnp.sum(xb * xb, axis=0, keepdims=True)
    d = d + xn
    lane = lax.broadcasted_iota(jnp.int32, (_S_TILE, N), 1)
    kio = lax.broadcasted_iota(jnp.int32, (_S_TILE, K), 1)
    acc = jnp.zeros((_S_TILE, K), jnp.int32)
    for k in range(K):
        mv = jnp.min(d, axis=1, keepdims=True)
        cand = jnp.where(d == mv, lane, _I32_MAX)
        ik = jnp.min(cand, axis=1, keepdims=True)
        acc = jnp.where(kio == k, ik, acc)
        d = jnp.where(lane == ik, float("inf"), d)
    b = pl.program_id(0)
    out_ref[0] = acc + b * N


def _knn_call(new_xyz, xyz):
    return pl.pallas_call(
        _knn_body,
        grid=(B, S // _S_TILE),
        in_specs=[
            pl.BlockSpec((1, 3, _S_TILE), lambda b, j: (b, 0, j)),
            pl.BlockSpec((1, 3, N), lambda b, j: (b, 0, 0)),
        ],
        out_specs=pl.BlockSpec((1, _S_TILE, K), lambda b, j: (b, j, 0)),
        out_shape=jax.ShapeDtypeStruct((B, S, K), jnp.int32),
    )(new_xyz, xyz)


_NROWS = B * S * K
_CHUNK = 128


def _sc_gather(proj, idx2d):
    info = plsc.get_sparse_core_info()
    nw = info.num_cores * info.num_subcores
    rows_per_w = _NROWS // nw
    nchunk = rows_per_w // _CHUNK
    mesh = plsc.VectorSubcoreMesh(core_axis_name="c", subcore_axis_name="s")

    @functools.partial(
        pl.kernel,
        mesh=mesh,
        out_type=jax.ShapeDtypeStruct((_NROWS, _CPAD), jnp.float32),
        scratch_types=[
            pltpu.VMEM((nchunk, _CHUNK), jnp.int32),
            pltpu.VMEM((_CHUNK, _CPAD), jnp.float32),
            pltpu.VMEM((_CHUNK, _CPAD), jnp.float32),
            pltpu.SemaphoreType.DMA,
            pltpu.SemaphoreType.DMA,
        ],
    )
    def gk(table_hbm, idx_hbm, out_hbm, idx_v, buf0, buf1, sem0, sem1):
        wid = lax.axis_index("s") * info.num_cores + lax.axis_index("c")
        pltpu.sync_copy(idx_hbm.at[pl.ds(wid * nchunk, nchunk)], idx_v)
        bufs = (buf0, buf1)
        sems = (sem0, sem1)
        cps = [pltpu.async_copy(table_hbm.at[idx_v.at[0]], buf0, sem0)]
        for j in range(nchunk):
            cps[j].wait()
            if j + 1 < nchunk:
                cps.append(pltpu.async_copy(table_hbm.at[idx_v.at[j + 1]],
                                            bufs[(j + 1) % 2], sems[(j + 1) % 2]))
            pltpu.sync_copy(bufs[j % 2],
                            out_hbm.at[pl.ds(wid * rows_per_w + j * _CHUNK, _CHUNK)])

    return gk(proj, idx2d)


_T_TILE = 512


def _tail_body(g_ref, nq_ref, w3_ref, wcc_ref, wcp_ref, wpc_ref, wpp_ref,
               wwp_ref, wwc_ref, out_ref):
    def dg(a, b_):
        return lax.dot_general(a, b_, (((1,), (1,)), ((), ())),
                               preferred_element_type=jnp.float32)

    g = g_ref[...]
    nq = nq_ref[0]
    pq = lax.dot_general(nq, w3_ref[...], (((0,), (1,)), ((), ())),
                         preferred_element_type=jnp.float32)
    x = _leaky(g - pq[:, None, :])
    ca = jnp.sum(x, axis=1) * (1.0 / K)
    pa = jnp.sum(x, axis=2) * (1.0 / OUT)
    ac = _leaky(dg(ca, wcc_ref[...]) + dg(pa, wcp_ref[...]))
    ap = _leaky(dg(ca, wpc_ref[...]) + dg(pa, wpp_ref[...]))
    wp = 1.0 / (1.0 + jnp.exp(-dg(ap, wwp_ref[...])))
    wc = 1.0 / (1.0 + jnp.exp(-dg(ac, wwc_ref[...])))
    s1 = jnp.sum(x * wp[:, :, None], axis=1)
    s1t = lax.transpose(s1, (1, 0))[:OUT]
    wct = lax.transpose(wc, (1, 0))
    out_ref[0] = s1t * wct * (1.0 / K)


def _tail_call(grouped, new_xyz, W3pad, Wcc, Wcp, Wpc, Wpp, W_wp, W_wc):
    nt = (B * S) // _T_TILE
    per_b = S // _T_TILE
    return pl.pallas_call(
        _tail_body,
        grid=(nt,),
        in_specs=[
            pl.BlockSpec((_T_TILE, K, _CPAD), lambda i: (i, 0, 0)),
            pl.BlockSpec((1, 3, _T_TILE), lambda i: (i // per_b, 0, i % per_b)),
            pl.BlockSpec((_CPAD, 3), lambda i: (0, 0)),
            pl.BlockSpec((OUT, _CPAD), lambda i: (0, 0)),
            pl.BlockSpec((OUT, K), lambda i: (0, 0)),
            pl.BlockSpec((K, _CPAD), lambda i: (0, 0)),
            pl.BlockSpec((K, K), lambda i: (0, 0)),
            pl.BlockSpec((K, K), lambda i: (0, 0)),
            pl.BlockSpec((OUT, OUT), lambda i: (0, 0)),
        ],
        out_specs=pl.BlockSpec((1, OUT, _T_TILE),
                               lambda i: (i // per_b, 0, i % per_b)),
        out_shape=jax.ShapeDtypeStruct((B, OUT, S), jnp.float32),
    )(grouped, new_xyz, W3pad, Wcc, Wcp, Wpc, Wpp, W_wp, W_wc)


def kernel(xyz, points, W_kernel, W_linear, W_wp, W_wc):
    pad = ((0, _CPAD - OUT), (0, 0))
    W3pad = jnp.pad(W_kernel[:, :3], pad)
    Wppad = jnp.pad(W_kernel[:, 3:], pad)
    Wcc = jnp.pad(W_linear[:OUT, :OUT], ((0, 0), (0, _CPAD - OUT)))
    Wcp = W_linear[:OUT, OUT:]
    Wpc = jnp.pad(W_linear[OUT:, :OUT], ((0, 0), (0, _CPAD - OUT)))
    Wpp = W_linear[OUT:, OUT:]

    fps_idx, new_xyz = _fps_call(xyz)
    proj = _proj_call(xyz, points, W3pad, Wppad)
    knn_gidx = _knn_call(new_xyz, xyz)
    grouped = _sc_gather(proj, knn_gidx.reshape(_NROWS // _CHUNK, _CHUNK))
    out = _tail_call(grouped.reshape(B * S, K, _CPAD), new_xyz,
                     W3pad, Wcc, Wcp, Wpc, Wpp, W_wp, W_wc)
    return (new_xyz, out, fps_idx)


# speedup vs baseline: 20.2325x; 1.0300x over previous
import functools

import jax
import jax.numpy as jnp
from jax import lax
from jax.experimental import pallas as pl
from jax.experimental.pallas import tpu as pltpu
from jax.experimental.pallas import tpu_sc as plsc

B = 2
N = 8192
S = 2048
K = 16
D = 64
OUT = 64
LEAKY = 0.1

_I32_MAX = 2147483647


def _leaky(x):
    return jnp.where(x >= 0, x, LEAKY * x)


def _fps_body(xyz_ref, idx_ref, nxyz_ref):
    x = xyz_ref[:, 0]
    y = xyz_ref[:, 1]
    z = xyz_ref[:, 2]
    lin = jnp.broadcast_to(
        (lax.broadcasted_iota(jnp.int32, (64, 128), 0) * 128
         + lax.broadcasted_iota(jnp.int32, (64, 128), 1))[None],
        (B, 64, 128))
    lin2 = (lax.broadcasted_iota(jnp.int32, (16, 128), 0) * 128
            + lax.broadcasted_iota(jnp.int32, (16, 128), 1))[None]

    def amax5(v, i, a0, a1, a2):
        vals = [v, i, a0, a1, a2]
        for h in (32, 16, 8):
            va, vb = vals[0][:, :h], vals[0][:, h:]
            c = va >= vb
            vals = [jnp.where(c, t[:, :h], t[:, h:]) for t in vals]
        v8, i8, x8, y8, z8 = vals

        def red2(a, op):
            return op(op(a, axis=2, keepdims=True), axis=1, keepdims=True)

        m = red2(v8, jnp.max)
        cand = jnp.where(v8 == m, i8, _I32_MAX)
        far = red2(cand, jnp.min)
        em = i8 == far
        cx = red2(jnp.where(em, x8, 0.0), jnp.sum)
        cy = red2(jnp.where(em, y8, 0.0), jnp.sum)
        cz = red2(jnp.where(em, z8, 0.0), jnp.sum)
        return m, far, cx, cy, cz

    def body(it, carry):
        far, cx, cy, cz, dist = carry
        m2 = lin2 == it
        idx_ref[...] = jnp.where(m2, far, idx_ref[...])
        nxyz_ref[:, 0] = jnp.where(m2, cx, nxyz_ref[:, 0])
        nxyz_ref[:, 1] = jnp.where(m2, cy, nxyz_ref[:, 1])
        nxyz_ref[:, 2] = jnp.where(m2, cz, nxyz_ref[:, 2])
        dx = x - cx
        dy = y - cy
        dz = z - cz
        d = dx * dx + dy * dy
        d = d + dz * dz
        dist = jnp.minimum(dist, d)
        _, far_new, ncx, ncy, ncz = amax5(dist, lin, x, y, z)
        return far_new, ncx, ncy, ncz, dist

    far0 = jnp.zeros((B, 1, 1), jnp.int32)
    cx0 = x[:, 0:1, 0:1]
    cy0 = y[:, 0:1, 0:1]
    cz0 = z[:, 0:1, 0:1]
    dist0 = jnp.full((B, 64, 128), 1e10, jnp.float32)
    lax.fori_loop(0, S, body, (far0, cx0, cy0, cz0, dist0))


def _fps_call(xyz):
    xyzr = xyz.reshape(B, 3, 64, 128)
    idx, nxyz = pl.pallas_call(
        _fps_body,
        out_shape=[
            jax.ShapeDtypeStruct((B, 16, 128), jnp.int32),
            jax.ShapeDtypeStruct((B, 3, 16, 128), jnp.float32),
        ],
    )(xyzr)
    return idx.reshape(B, S), nxyz.reshape(B, 3, S)


_CPAD = 128


def _proj_body(xyz_ref, pts_ref, w3_ref, wp_ref, out_ref):
    xb = xyz_ref[0]
    pb = pts_ref[0]
    pt = (lax.dot_general(w3_ref[...], xb, (((1,), (0,)), ((), ())),
                          preferred_element_type=jnp.float32)
          + lax.dot_general(wp_ref[...], pb, (((1,), (0,)), ((), ())),
                            preferred_element_type=jnp.float32))
    out_ref[...] = lax.transpose(pt, (1, 0))


def _proj_call(xyz, points, W3pad, Wppad):
    return pl.pallas_call(
        _proj_body,
        grid=(B,),
        in_specs=[
            pl.BlockSpec((1, 3, N), lambda b: (b, 0, 0)),
            pl.BlockSpec((1, D, N), lambda b: (b, 0, 0)),
            pl.BlockSpec((_CPAD, 3), lambda b: (0, 0)),
            pl.BlockSpec((_CPAD, D), lambda b: (0, 0)),
        ],
        out_specs=pl.BlockSpec((N, _CPAD), lambda b: (b, 0)),
        out_shape=jax.ShapeDtypeStruct((B * N, _CPAD), jnp.float32),
    )(xyz, points, W3pad, Wppad)


_S_TILE = 128


def _knn_body(nq_ref, xyz_ref, out_ref):
    q = nq_ref[0]
    xb = xyz_ref[0]
    d = -2.0 * lax.dot_general(q, xb, (((0,), (0,)), ((), ())),
                               preferred_element_type=jnp.float32)
    qn = lax.dot_general(q * q, jnp.ones((3, 1), jnp.float32),
                         (((0,), (0,)), ((), ())),
                         preferred_element_type=jnp.float32)
    d = d + qn
    xn = jnp.sum(xb * xb, axis=0, keepdims=True)
    d = d + xn
    lane = lax.broadcasted_iota(jnp.int32, (_S_TILE, N), 1)
    kio = lax.broadcasted_iota(jnp.int32, (_S_TILE, K), 1)
    acc = jnp.zeros((_S_TILE, K), jnp.int32)
    for k in range(K):
        mv = jnp.min(d, axis=1, keepdims=True)
        cand = jnp.where(d == mv, lane, _I32_MAX)
        ik = jnp.min(cand, axis=1, keepdims=True)
        acc = jnp.where(kio == k, ik, acc)
        d = jnp.where(lane == ik, float("inf"), d)
    b = pl.program_id(0)
    out_ref[0] = acc + b * N


def _knn_call(new_xyz, xyz):
    return pl.pallas_call(
        _knn_body,
        grid=(B, S // _S_TILE),
        in_specs=[
            pl.BlockSpec((1, 3, _S_TILE), lambda b, j: (b, 0, j)),
            pl.BlockSpec((1, 3, N), lambda b, j: (b, 0, 0)),
        ],
        out_specs=pl.BlockSpec((1, _S_TILE, K), lambda b, j: (b, j, 0)),
        out_shape=jax.ShapeDtypeStruct((B, S, K), jnp.int32),
    )(new_xyz, xyz)


_NROWS = B * S * K
_CHUNK = 128


def _sc_gather(proj, idx2d):
    info = plsc.get_sparse_core_info()
    nw = info.num_cores * info.num_subcores
    rows_per_w = _NROWS // nw
    nchunk = rows_per_w // _CHUNK
    mesh = plsc.VectorSubcoreMesh(core_axis_name="c", subcore_axis_name="s")

    @functools.partial(
        pl.kernel,
        mesh=mesh,
        out_type=jax.ShapeDtypeStruct((_NROWS, _CPAD), jnp.float32),
        scratch_types=[
            pltpu.VMEM((nchunk, _CHUNK), jnp.int32),
            pltpu.VMEM((_CHUNK, _CPAD), jnp.float32),
            pltpu.VMEM((_CHUNK, _CPAD), jnp.float32),
            pltpu.SemaphoreType.DMA,
            pltpu.SemaphoreType.DMA,
        ],
    )
    def gk(table_hbm, idx_hbm, out_hbm, idx_v, buf0, buf1, sem0, sem1):
        wid = lax.axis_index("s") * info.num_cores + lax.axis_index("c")
        pltpu.sync_copy(idx_hbm.at[pl.ds(wid * nchunk, nchunk)], idx_v)
        bufs = (buf0, buf1)
        sems = (sem0, sem1)
        cps = [pltpu.async_copy(table_hbm.at[idx_v.at[0]], buf0, sem0)]
        for j in range(nchunk):
            cps[j].wait()
            if j + 1 < nchunk:
                cps.append(pltpu.async_copy(table_hbm.at[idx_v.at[j + 1]],
                                            bufs[(j + 1) % 2], sems[(j + 1) % 2]))
            pltpu.sync_copy(bufs[j % 2],
                            out_hbm.at[pl.ds(wid * rows_per_w + j * _CHUNK, _CHUNK)])

    return gk(proj, idx2d)


_T_TILE = 512


def _tail_body(g_ref, nq_ref, w3_ref, wcc_ref, wcp_ref, wpc_ref, wpp_ref,
               wwp_ref, wwc_ref, out_ref):
    def dg(a, b_):
        return lax.dot_general(a, b_, (((1,), (1,)), ((), ())),
                               preferred_element_type=jnp.float32)

    g = g_ref[...]
    nq = nq_ref[0]
    pq = lax.dot_general(nq, w3_ref[...], (((0,), (1,)), ((), ())),
                         preferred_element_type=jnp.float32)
    x = _leaky(g - pq[:, None, :])
    ca = jnp.sum(x, axis=1) * (1.0 / K)
    pa = jnp.sum(x, axis=2) * (1.0 / OUT)
    ac = _leaky(dg(ca, wcc_ref[...]) + dg(pa, wcp_ref[...]))
    ap = _leaky(dg(ca, wpc_ref[...]) + dg(pa, wpp_ref[...]))
    wp = 1.0 / (1.0 + jnp.exp(-dg(ap, wwp_ref[...])))
    wc = 1.0 / (1.0 + jnp.exp(-dg(ac, wwc_ref[...])))
    s1 = jnp.sum(x * wp[:, :, None], axis=1)
    s1t = lax.transpose(s1, (1, 0))[:OUT]
    wct = lax.transpose(wc, (1, 0))
    out_ref[0] = s1t * wct * (1.0 / K)


def _tail_call(grouped, new_xyz, W3pad, Wcc, Wcp, Wpc, Wpp, W_wp, W_wc):
    nt = (B * S) // _T_TILE
    per_b = S // _T_TILE
    return pl.pallas_call(
        _tail_body,
        grid=(nt,),
        in_specs=[
            pl.BlockSpec((_T_TILE, K, _CPAD), lambda i: (i, 0, 0)),
            pl.BlockSpec((1, 3, _T_TILE), lambda i: (i // per_b, 0, i % per_b)),
            pl.BlockSpec((_CPAD, 3), lambda i: (0, 0)),
            pl.BlockSpec((OUT, _CPAD), lambda i: (0, 0)),
            pl.BlockSpec((OUT, K), lambda i: (0, 0)),
            pl.BlockSpec((K, _CPAD), lambda i: (0, 0)),
            pl.BlockSpec((K, K), lambda i: (0, 0)),
            pl.BlockSpec((K, K), lambda i: (0, 0)),
            pl.BlockSpec((OUT, OUT), lambda i: (0, 0)),
        ],
        out_specs=pl.BlockSpec((1, OUT, _T_TILE),
                               lambda i: (i // per_b, 0, i % per_b)),
        out_shape=jax.ShapeDtypeStruct((B, OUT, S), jnp.float32),
    )(grouped, new_xyz, W3pad, Wcc, Wcp, Wpc, Wpp, W_wp, W_wc)


def kernel(xyz, points, W_kernel, W_linear, W_wp, W_wc):
    pad = ((0, _CPAD - OUT), (0, 0))
    W3pad = jnp.pad(W_kernel[:, :3], pad)
    Wppad = jnp.pad(W_kernel[:, 3:], pad)
    Wcc = jnp.pad(W_linear[:OUT, :OUT], ((0, 0), (0, _CPAD - OUT)))
    Wcp = W_linear[:OUT, OUT:]
    Wpc = jnp.pad(W_linear[OUT:, :OUT], ((0, 0), (0, _CPAD - OUT)))
    Wpp = W_linear[OUT:, OUT:]

    fps_idx, new_xyz = _fps_call(xyz)
    proj = _proj_call(xyz, points, W3pad, Wppad)
    knn_gidx = _knn_call(new_xyz, xyz)
    grouped = _sc_gather(proj, knn_gidx.reshape(_NROWS // _CHUNK, _CHUNK))
    out = _tail_call(grouped.reshape(B * S, K, _CPAD), new_xyz,
                     W3pad, Wcc, Wcp, Wpc, Wpp, W_wp, W_wc)
    return (new_xyz, out, fps_idx)
